# baseline (speedup 1.0000x reference)
import functools

import jax
import jax.numpy as jnp
from jax import lax
from jax.experimental import pallas as pl
from jax.experimental.pallas import tpu as pltpu
from jax.experimental.pallas import tpu_sc as plsc

NC = 2
NS = 16
NW = NC * NS
LANES = 16
K = 128


def _make_deg_kernel(n, ch):
    zn = n // LANES
    mesh = plsc.VectorSubcoreMesh(core_axis_name="c", subcore_axis_name="s")

    @functools.partial(
        pl.kernel,
        out_type=jax.ShapeDtypeStruct((NW, n), jnp.float32),
        mesh=mesh,
        compiler_params=pltpu.CompilerParams(needs_layout_passes=False),
        scratch_types=[
            pltpu.VMEM((n,), jnp.float32),
            pltpu.VMEM((ch, K), jnp.int32),
            pltpu.VMEM((ch, K), jnp.float32),
        ],
    )
    def deg_kernel(col_hbm, w_hbm, out_hbm, dacc, cbuf, wbuf):
        c = lax.axis_index("c")
        s = lax.axis_index("s")
        wid = c * NS + s

        def zero_body(i, _):
            dacc[pl.ds(i * LANES, LANES)] = jnp.zeros((LANES,), jnp.float32)
            return 0

        lax.fori_loop(0, zn, zero_body, 0)

        pltpu.sync_copy(col_hbm.at[wid], cbuf)
        pltpu.sync_copy(w_hbm.at[wid], wbuf)

        def chunk_body(t, _):
            def grp_body(g, _):
                cv = cbuf[t, pl.ds(g * LANES, LANES)]
                wg = wbuf[t, pl.ds(g * LANES, LANES)]
                plsc.addupdate_scatter(dacc, [cv], wg)
                return 0

            lax.fori_loop(0, K // LANES, grp_body, 0)
            return 0

        lax.fori_loop(0, ch, chunk_body, 0)
        pltpu.sync_copy(dacc, out_hbm.at[wid])

    return deg_kernel


def _make_agg_kernel(n, ch, d):
    rows_per_tile = n // NS
    zr = rows_per_tile // 5
    nsl = d // LANES
    mesh = plsc.VectorSubcoreMesh(core_axis_name="c", subcore_axis_name="s")

    @functools.partial(
        pl.kernel,
        out_type=jax.ShapeDtypeStruct((NC, n, d), jnp.float32),
        mesh=mesh,
        compiler_params=pltpu.CompilerParams(
            needs_layout_passes=False, use_tc_tiling_on_sc=False),
        scratch_types=[
            pltpu.VMEM((ch, K), jnp.int32),
            pltpu.VMEM((ch, K), jnp.int32),
            pltpu.VMEM((ch, K), jnp.float32),
            pltpu.VMEM((K, d), jnp.float32),
            pltpu.VMEM((K, d), jnp.float32),
            pltpu.VMEM((K, d), jnp.float32),
            pltpu.VMEM((K, d), jnp.float32),
            pltpu.VMEM((K, d), jnp.float32),
            pltpu.VMEM((K, d), jnp.float32),
            pltpu.VMEM((zr, d), jnp.float32),
            pltpu.VMEM_SHARED((n, d), jnp.float32),
            pltpu.SemaphoreType.DMA,
            pltpu.SemaphoreType.DMA,
            pltpu.SemaphoreType.DMA,
            pltpu.SemaphoreType.DMA,
            pltpu.SemaphoreType.DMA,
            pltpu.SemaphoreType.DMA,
        ],
    )
    def agg_kernel(hp_hbm, row_hbm, col_hbm, w_hbm, out_hbm,
                   rbuf, cbuf, wbuf, g0, g1, g2, s0, s1, s2, zbuf, acc,
                   gsem0, gsem1, gsem2, ssem0, ssem1, ssem2):
        c = lax.axis_index("c")
        s = lax.axis_index("s")
        wid = c * NS + s

        def zb_body(i, _):
            for j in range(nsl):
                zbuf[i, pl.ds(j * LANES, LANES)] = jnp.zeros((LANES,),
                                                             jnp.float32)
            return 0

        lax.fori_loop(0, zr, zb_body, 0)
        rbase = s * rows_per_tile
        for q in range(rows_per_tile // zr):
            pltpu.sync_copy(zbuf, acc.at[pl.ds(rbase + q * zr, zr)])

        pltpu.sync_copy(row_hbm.at[wid], rbuf)
        pltpu.sync_copy(col_hbm.at[wid], cbuf)
        pltpu.sync_copy(w_hbm.at[wid], wbuf)

        plsc.subcore_barrier()

        def scale(t, gb, sb):
            def grp_body(g, _):
                wv = wbuf[t, pl.ds(g * LANES, LANES)]
                for e in range(LANES):
                    wb = wv.at[jnp.full((LANES,), e, jnp.int32)].get(
                        mode="promise_in_bounds")
                    i = g * LANES + e
                    for j in range(nsl):
                        sl = pl.ds(j * LANES, LANES)
                        sb[i, sl] = gb[i, sl] * wb
                return 0

            lax.fori_loop(0, K // LANES, grp_body, 0)

        nb = 3
        bufs = ((g0, s0, gsem0, ssem0),
                (g1, s1, gsem1, ssem1),
                (g2, s2, gsem2, ssem2))

        def slot(t, tp, gb, sb, gsem, ssem):
            pltpu.make_async_copy(hp_hbm.at[rbuf.at[t]], gb, gsem).wait()

            @pl.when(tp >= 1)
            def _():
                pltpu.make_async_copy(sb, acc.at[cbuf.at[t]], ssem).wait()

            scale(t, gb, sb)

            @pl.when(t + nb <= ch - 1)
            def _():
                pltpu.async_copy(hp_hbm.at[rbuf.at[t + nb]], gb, gsem)

            pltpu.async_copy(sb, acc.at[cbuf.at[t]], ssem, add=True)

        for b in range(nb):
            gb, sb, gsem, ssem = bufs[b]
            pltpu.async_copy(hp_hbm.at[rbuf.at[b]], gb, gsem)

        def ring_body(tp, _):
            for b in range(nb):
                gb, sb, gsem, ssem = bufs[b]
                slot(nb * tp + b, tp, gb, sb, gsem, ssem)
            return 0

        full = ch // nb
        lax.fori_loop(0, full, ring_body, 0)

        for r in range(ch % nb):
            t = nb * full + r
            gb, sb, gsem, ssem = bufs[r]
            pltpu.make_async_copy(hp_hbm.at[rbuf.at[t]], gb, gsem).wait()
            pltpu.make_async_copy(sb, acc.at[cbuf.at[t - nb]], ssem).wait()
            scale(t, gb, sb)
            pltpu.async_copy(sb, acc.at[cbuf.at[t]], ssem, add=True)

        for b in range(nb):
            t = ch - nb + b
            gb, sb, gsem, ssem = bufs[t % nb]
            pltpu.make_async_copy(sb, acc.at[cbuf.at[t]], ssem).wait()

        plsc.subcore_barrier()
        pltpu.sync_copy(acc.at[pl.ds(rbase, rows_per_tile)],
                        out_hbm.at[c, pl.ds(rbase, rows_per_tile)])

    return agg_kernel


def _tc_pre_body(degp_ref, x_ref, w_ref, hp_ref, dinv_ref):
    deg = jnp.sum(degp_ref[...], axis=0)
    dinv = jnp.where(deg > 0, lax.rsqrt(jnp.where(deg > 0, deg, 1.0)), 0.0)
    h = jnp.dot(x_ref[...], w_ref[...], preferred_element_type=jnp.float32)
    hp_ref[...] = dinv[:, None] * h
    dinv_ref[...] = dinv


def _tc_mid_body(s_ref, dinv_ref, b_ref, w_ref, h_ref, hp_ref):
    dinv = dinv_ref[...]
    agg = s_ref[0] + s_ref[1]
    h = jax.nn.relu(dinv[:, None] * agg + b_ref[...][None, :])
    h_ref[...] = h
    hp_ref[...] = dinv[:, None] * jnp.dot(
        h, w_ref[...], preferred_element_type=jnp.float32)


def _tc_fin_body(s_ref, dinv_ref, b_ref, h_ref):
    dinv = dinv_ref[...]
    agg = s_ref[0] + s_ref[1]
    h_ref[...] = dinv[:, None] * agg + b_ref[...][None, :]


def kernel(x, edge_index, edge_weight, W1, b1, W2, b2, W3, b3):
    n, d_in = x.shape
    e = edge_index.shape[1]
    d_h = W1.shape[1]
    d_out = W3.shape[1]
    per_tile = e // NW
    ch = -(-per_tile // K)
    pad = ch * K - per_tile

    rowm = edge_index[0].reshape(NW, per_tile)
    colm = edge_index[1].reshape(NW, per_tile)
    wm = edge_weight.reshape(NW, per_tile)
    if pad:
        rowm = jnp.pad(rowm, ((0, 0), (0, pad)))
        colm = jnp.pad(colm, ((0, 0), (0, pad)))
        wm = jnp.pad(wm, ((0, 0), (0, pad)))
    row = rowm.reshape(NW, ch, K)
    col = colm.reshape(NW, ch, K)
    w3d = wm.reshape(NW, ch, K)

    deg_kernel = _make_deg_kernel(n, ch)
    degp = deg_kernel(col, w3d)

    hp1, dinv = pl.pallas_call(
        _tc_pre_body,
        out_shape=[
            jax.ShapeDtypeStruct((n, d_h), jnp.float32),
            jax.ShapeDtypeStruct((n,), jnp.float32),
        ],
    )(degp, x, W1)

    agg = _make_agg_kernel(n, ch, d_h)

    s1 = agg(hp1, row, col, w3d)
    h1, hp2 = pl.pallas_call(
        _tc_mid_body,
        out_shape=[
            jax.ShapeDtypeStruct((n, d_h), jnp.float32),
            jax.ShapeDtypeStruct((n, W2.shape[1]), jnp.float32),
        ],
    )(s1, dinv, b1, W2)

    s2 = agg(hp2, row, col, w3d)
    h2, hp3 = pl.pallas_call(
        _tc_mid_body,
        out_shape=[
            jax.ShapeDtypeStruct((n, d_h), jnp.float32),
            jax.ShapeDtypeStruct((n, W3.shape[1]), jnp.float32),
        ],
    )(s2, dinv, b2, W3)

    s3 = agg(hp3, row, col, w3d)
    h3 = pl.pallas_call(
        _tc_fin_body,
        out_shape=jax.ShapeDtypeStruct((n, d_out), jnp.float32),
    )(s3, dinv, b3)

    return jnp.stack([h1, h2, h3], axis=0)


# speedup vs baseline: 1.5853x; 1.5853x over previous
import functools

import jax
import jax.numpy as jnp
from jax import lax
from jax.experimental import pallas as pl
from jax.experimental.pallas import tpu as pltpu
from jax.experimental.pallas import tpu_sc as plsc

NC = 2
NS = 16
NW = NC * NS
LANES = 16
K = 80


def _make_deg_kernel(n, e):
    per_tile = e // NW
    ch = per_tile // K
    zn = n // LANES
    mesh = plsc.VectorSubcoreMesh(core_axis_name="c", subcore_axis_name="s")

    @functools.partial(
        pl.kernel,
        out_type=jax.ShapeDtypeStruct((NW, n), jnp.float32),
        mesh=mesh,
        compiler_params=pltpu.CompilerParams(needs_layout_passes=False),
        scratch_types=[
            pltpu.VMEM((n,), jnp.float32),
            pltpu.VMEM((ch, K), jnp.int32),
            pltpu.VMEM((ch, K), jnp.float32),
        ],
    )
    def deg_kernel(col_hbm, w_hbm, out_hbm, dacc, cbuf, wbuf):
        c = lax.axis_index("c")
        s = lax.axis_index("s")
        wid = c * NS + s

        def zero_body(i, _):
            dacc[pl.ds(i * LANES, LANES)] = jnp.zeros((LANES,), jnp.float32)
            return 0

        lax.fori_loop(0, zn, zero_body, 0)

        pltpu.sync_copy(col_hbm.at[wid], cbuf)
        pltpu.sync_copy(w_hbm.at[wid], wbuf)

        def chunk_body(t, _):
            def grp_body(g, _):
                cv = cbuf[t, pl.ds(g * LANES, LANES)]
                wg = wbuf[t, pl.ds(g * LANES, LANES)]
                plsc.addupdate_scatter(dacc, [cv], wg)
                return 0

            lax.fori_loop(0, K // LANES, grp_body, 0)
            return 0

        lax.fori_loop(0, ch, chunk_body, 0)
        pltpu.sync_copy(dacc, out_hbm.at[wid])

    return deg_kernel


def _make_agg_kernel(n, e, d):
    per_tile = e // NW
    ch = per_tile // K
    rows_per_tile = n // NS
    zr = rows_per_tile // 5
    nsl = d // LANES
    mesh = plsc.VectorSubcoreMesh(core_axis_name="c", subcore_axis_name="s")

    @functools.partial(
        pl.kernel,
        out_type=jax.ShapeDtypeStruct((NC, n, d), jnp.float32),
        mesh=mesh,
        compiler_params=pltpu.CompilerParams(
            needs_layout_passes=False, use_tc_tiling_on_sc=False),
        scratch_types=[
            pltpu.VMEM((ch, K), jnp.int32),
            pltpu.VMEM((ch, K), jnp.int32),
            pltpu.VMEM((ch, K), jnp.float32),
            pltpu.VMEM((K, d), jnp.float32),
            pltpu.VMEM((K, d), jnp.float32),
            pltpu.VMEM((K, d), jnp.float32),
            pltpu.VMEM((K, d), jnp.float32),
            pltpu.VMEM((K, d), jnp.float32),
            pltpu.VMEM((K, d), jnp.float32),
            pltpu.VMEM((zr, d), jnp.float32),
            pltpu.VMEM_SHARED((n, d), jnp.float32),
            pltpu.SemaphoreType.DMA,
            pltpu.SemaphoreType.DMA,
            pltpu.SemaphoreType.DMA,
            pltpu.SemaphoreType.DMA,
            pltpu.SemaphoreType.DMA,
            pltpu.SemaphoreType.DMA,
        ],
    )
    def agg_kernel(hp_hbm, row_hbm, col_hbm, w_hbm, out_hbm,
                   rbuf, cbuf, wbuf, g0, g1, g2, s0, s1, s2, zbuf, acc,
                   gsem0, gsem1, gsem2, ssem0, ssem1, ssem2):
        c = lax.axis_index("c")
        s = lax.axis_index("s")
        wid = c * NS + s

        def zb_body(i, _):
            for j in range(nsl):
                zbuf[i, pl.ds(j * LANES, LANES)] = jnp.zeros((LANES,),
                                                             jnp.float32)
            return 0

        lax.fori_loop(0, zr, zb_body, 0)
        rbase = s * rows_per_tile
        for q in range(rows_per_tile // zr):
            pltpu.sync_copy(zbuf, acc.at[pl.ds(rbase + q * zr, zr)])

        pltpu.sync_copy(row_hbm.at[wid], rbuf)
        pltpu.sync_copy(col_hbm.at[wid], cbuf)
        pltpu.sync_copy(w_hbm.at[wid], wbuf)

        plsc.subcore_barrier()

        def scale(t, gb, sb):
            def grp_body(g, _):
                wv = wbuf[t, pl.ds(g * LANES, LANES)]
                for e in range(LANES):
                    wb = wv.at[jnp.full((LANES,), e, jnp.int32)].get(
                        mode="promise_in_bounds")
                    i = g * LANES + e
                    for j in range(nsl):
                        sl = pl.ds(j * LANES, LANES)
                        sb[i, sl] = gb[i, sl] * wb
                return 0

            lax.fori_loop(0, K // LANES, grp_body, 0)

        nb = 3
        bufs = ((g0, s0, gsem0, ssem0),
                (g1, s1, gsem1, ssem1),
                (g2, s2, gsem2, ssem2))

        def slot(t, tp, gb, sb, gsem, ssem):
            pltpu.make_async_copy(hp_hbm.at[rbuf.at[t]], gb, gsem).wait()

            @pl.when(tp >= 1)
            def _():
                pltpu.make_async_copy(sb, acc.at[cbuf.at[t]], ssem).wait()

            scale(t, gb, sb)

            @pl.when(t + nb <= ch - 1)
            def _():
                pltpu.async_copy(hp_hbm.at[rbuf.at[t + nb]], gb, gsem)

            pltpu.async_copy(sb, acc.at[cbuf.at[t]], ssem, add=True)

        for b in range(nb):
            gb, sb, gsem, ssem = bufs[b]
            pltpu.async_copy(hp_hbm.at[rbuf.at[b]], gb, gsem)

        def ring_body(tp, _):
            for b in range(nb):
                gb, sb, gsem, ssem = bufs[b]
                slot(nb * tp + b, tp, gb, sb, gsem, ssem)
            return 0

        full = ch // nb
        lax.fori_loop(0, full, ring_body, 0)

        for r in range(ch % nb):
            t = nb * full + r
            gb, sb, gsem, ssem = bufs[r]
            pltpu.make_async_copy(hp_hbm.at[rbuf.at[t]], gb, gsem).wait()
            pltpu.make_async_copy(sb, acc.at[cbuf.at[t - nb]], ssem).wait()
            scale(t, gb, sb)
            pltpu.async_copy(sb, acc.at[cbuf.at[t]], ssem, add=True)

        for b in range(nb):
            t = ch - nb + b
            gb, sb, gsem, ssem = bufs[t % nb]
            pltpu.make_async_copy(sb, acc.at[cbuf.at[t]], ssem).wait()

        plsc.subcore_barrier()
        pltpu.sync_copy(acc.at[pl.ds(rbase, rows_per_tile)],
                        out_hbm.at[c, pl.ds(rbase, rows_per_tile)])

    return agg_kernel


def _tc_pre_body(degp_ref, x_ref, w_ref, hp_ref, dinv_ref):
    deg = jnp.sum(degp_ref[...], axis=0)
    dinv = jnp.where(deg > 0, lax.rsqrt(jnp.where(deg > 0, deg, 1.0)), 0.0)
    h = jnp.dot(x_ref[...], w_ref[...], preferred_element_type=jnp.float32)
    hp_ref[...] = dinv[:, None] * h
    dinv_ref[...] = dinv


def _tc_mid_body(s_ref, dinv_ref, b_ref, w_ref, h_ref, hp_ref):
    dinv = dinv_ref[...]
    agg = s_ref[0] + s_ref[1]
    h = jax.nn.relu(dinv[:, None] * agg + b_ref[...][None, :])
    h_ref[...] = h
    hp_ref[...] = dinv[:, None] * jnp.dot(
        h, w_ref[...], preferred_element_type=jnp.float32)


def _tc_fin_body(s_ref, dinv_ref, b_ref, h_ref):
    dinv = dinv_ref[...]
    agg = s_ref[0] + s_ref[1]
    h_ref[...] = dinv[:, None] * agg + b_ref[...][None, :]


def kernel(x, edge_index, edge_weight, W1, b1, W2, b2, W3, b3):
    n, d_in = x.shape
    e = edge_index.shape[1]
    d_h = W1.shape[1]
    d_out = W3.shape[1]
    per_tile = e // NW
    ch = per_tile // K

    row = edge_index[0].reshape(NW, ch, K)
    col = edge_index[1].reshape(NW, ch, K)
    w3d = edge_weight.reshape(NW, ch, K)

    deg_kernel = _make_deg_kernel(n, e)
    degp = deg_kernel(col, w3d)

    hp1, dinv = pl.pallas_call(
        _tc_pre_body,
        out_shape=[
            jax.ShapeDtypeStruct((n, d_h), jnp.float32),
            jax.ShapeDtypeStruct((n,), jnp.float32),
        ],
    )(degp, x, W1)

    agg = _make_agg_kernel(n, e, d_h)

    s1 = agg(hp1, row, col, w3d)
    h1, hp2 = pl.pallas_call(
        _tc_mid_body,
        out_shape=[
            jax.ShapeDtypeStruct((n, d_h), jnp.float32),
            jax.ShapeDtypeStruct((n, W2.shape[1]), jnp.float32),
        ],
    )(s1, dinv, b1, W2)

    s2 = agg(hp2, row, col, w3d)
    h2, hp3 = pl.pallas_call(
        _tc_mid_body,
        out_shape=[
            jax.ShapeDtypeStruct((n, d_h), jnp.float32),
            jax.ShapeDtypeStruct((n, W3.shape[1]), jnp.float32),
        ],
    )(s2, dinv, b2, W3)

    s3 = agg(hp3, row, col, w3d)
    h3 = pl.pallas_call(
        _tc_fin_body,
        out_shape=jax.ShapeDtypeStruct((n, d_out), jnp.float32),
    )(s3, dinv, b3)

    return jnp.stack([h1, h2, h3], axis=0)
